# baseline (speedup 1.0000x reference)
import functools

import jax
import jax.numpy as jnp
import numpy as np
from jax import lax
from jax.experimental import pallas as pl
from jax.experimental.pallas import tpu as pltpu
from jax.experimental.pallas import tpu_sc as plsc

_SR = 0.5

_NC = 2
_NS = 16
_L = 16
_NW = _NC * _NS

_SIGN = np.int32(-2**31)


def _f32_to_key(v):
    y = plsc.bitcast(v, jnp.int32)
    return jnp.where(y < 0, ~y, y ^ _SIGN)


def _scan_hist(hist_ref, nbins, krem):
    nchunks = nbins // _L
    iota = lax.iota(jnp.int32, _L)

    def body(i, carry):
        acc, kr, found, bsel = carry
        j = nchunks - 1 - i
        bins = hist_ref[pl.ds(j * _L, _L)]
        c = plsc.cumsum(bins)
        total = jnp.max(c)
        excl = c - bins
        rhs = acc + total - krem
        cond = excl <= rhs
        p = jnp.max(plsc.all_reduce_population_count(cond))
        newly = jnp.logical_and(jnp.logical_not(found), p > 0)
        local = p - 1
        c_at = jnp.sum(jnp.where(iota == local, c, 0))
        count_above = acc + total - c_at
        bsel = jnp.where(newly, j * _L + local, bsel)
        kr = jnp.where(newly, krem - count_above, kr)
        found = jnp.logical_or(found, p > 0)
        acc = acc + total
        return acc, kr, found, bsel

    zero = jnp.int32(0)
    _, kr, _, bsel = lax.fori_loop(
        0, nchunks, body, (zero, krem, False, zero))
    return bsel, kr


def _zero_hist(hist_ref, nbins):
    zeros = jnp.zeros((_L,), jnp.int32)

    def body(i, _):
        hist_ref[pl.ds(i * _L, _L)] = zeros
        return 0

    lax.fori_loop(0, nbins // _L, body, 0)


def _sc_thresholds(x, rows, cols, k):
    nvec = cols // _L
    nbins = 2048
    mesh = plsc.VectorSubcoreMesh(core_axis_name="c", subcore_axis_name="s")
    rows_per_w = rows // _NW

    @functools.partial(
        pl.kernel,
        mesh=mesh,
        out_type=jax.ShapeDtypeStruct((_NW, _L), jnp.int32),
        scratch_types=[
            pltpu.VMEM((cols,), jnp.float32),
            pltpu.VMEM((cols,), jnp.int32),
            pltpu.VMEM((nbins,), jnp.int32),
            pltpu.VMEM((_L,), jnp.int32),
        ],
        compiler_params=pltpu.CompilerParams(needs_layout_passes=False),
    )
    def thresh_kernel(x_hbm, out_hbm, data_v, key_v, hist_v, thr_v):
        c = lax.axis_index("c")
        s = lax.axis_index("s")
        wid = s * _NC + c
        iota = lax.iota(jnp.int32, _L)
        ones = jnp.ones((_L,), jnp.int32)

        thr_v[...] = jnp.zeros((_L,), jnp.int32)

        for r in range(rows_per_w):
            row = wid * rows_per_w + r
            pltpu.sync_copy(x_hbm.at[row], data_v)

            _zero_hist(hist_v, nbins)

            def pass1(j, _):
                v = data_v[pl.ds(j * _L, _L)]
                kv = _f32_to_key(v)
                key_v[pl.ds(j * _L, _L)] = kv
                idx = lax.shift_right_logical(kv, 21)
                plsc.addupdate_scatter(hist_v, [idx], ones)
                return 0

            lax.fori_loop(0, nvec, pass1, 0)
            b1, krem = _scan_hist(hist_v, nbins, jnp.int32(k))

            _zero_hist(hist_v, nbins)

            def pass2(j, _):
                kv = key_v[pl.ds(j * _L, _L)]
                m = lax.shift_right_logical(kv, 21) == b1
                idx = lax.shift_right_logical(kv, 10) & 0x7FF
                plsc.addupdate_scatter(hist_v, [idx], ones, mask=m)
                return 0

            lax.fori_loop(0, nvec, pass2, 0)
            b2, krem = _scan_hist(hist_v, nbins, krem)

            _zero_hist(hist_v, 1024)
            pref = (b1 << 11) | b2

            def pass3(j, _):
                kv = key_v[pl.ds(j * _L, _L)]
                m = lax.shift_right_logical(kv, 10) == pref
                idx = kv & 0x3FF
                plsc.addupdate_scatter(hist_v, [idx], ones, mask=m)
                return 0

            lax.fori_loop(0, nvec, pass3, 0)
            b3, _ = _scan_hist(hist_v, 1024, krem)

            tkey = (pref << 10) | b3
            thr_v[...] = jnp.where(iota == r, tkey, thr_v[...])

        pltpu.sync_copy(thr_v, out_hbm.at[wid])

    return thresh_kernel(x)


def _mask_block(x_ref, t_ref, o_ref):
    x = x_ref[...]
    t = t_ref[...]
    o_ref[...] = jnp.where(x >= t, x, jnp.float32(0.0))


@jax.jit
def kernel(x):
    rows, cols = x.shape
    k = int(_SR * cols)

    tkeys = _sc_thresholds(x, rows, cols, k)
    rows_per_w = rows // _NW
    tkeys = tkeys[:, :rows_per_w].reshape(rows, 1)
    tbits = jnp.where(tkeys < 0, tkeys ^ _SIGN, ~tkeys)
    thr = lax.bitcast_convert_type(tbits, jnp.float32)

    blk = 16
    grid = (rows // blk,)
    return pl.pallas_call(
        _mask_block,
        grid=grid,
        in_specs=[
            pl.BlockSpec((blk, cols), lambda i: (i, 0)),
            pl.BlockSpec((blk, 1), lambda i: (i, 0)),
        ],
        out_specs=pl.BlockSpec((blk, cols), lambda i: (i, 0)),
        out_shape=jax.ShapeDtypeStruct((rows, cols), x.dtype),
    )(x, thr)


# speedup vs baseline: 1.0388x; 1.0388x over previous
import functools

import jax
import jax.numpy as jnp
import numpy as np
from jax import lax
from jax.experimental import pallas as pl
from jax.experimental.pallas import tpu as pltpu
from jax.experimental.pallas import tpu_sc as plsc

_SR = 0.5

_NC = 2
_NS = 16
_L = 16
_NW = _NC * _NS

_SIGN = np.int32(-2**31)


def _f32_to_key(v):
    y = plsc.bitcast(v, jnp.int32)
    return jnp.where(y < 0, ~y, y ^ _SIGN)


def _scan_hist(hist_ref, nbins, krem):
    nchunks = nbins // _L
    iota = lax.iota(jnp.int32, _L)

    acc = jnp.int32(0)
    kr = krem
    found = jnp.bool_(False)
    bsel = jnp.int32(0)
    for j in range(nchunks - 1, -1, -1):
        bins = hist_ref[pl.ds(j * _L, _L)]
        c = plsc.cumsum(bins)
        total = jnp.max(c)
        excl = c - bins
        rhs = acc + total - krem
        cond = excl <= rhs
        p = jnp.max(plsc.all_reduce_population_count(cond))
        newly = jnp.logical_and(jnp.logical_not(found), p > 0)
        local = p - 1
        c_at = jnp.sum(jnp.where(iota == local, c, 0))
        count_above = acc + total - c_at
        bsel = jnp.where(newly, j * _L + local, bsel)
        kr = jnp.where(newly, krem - count_above, kr)
        found = jnp.logical_or(found, p > 0)
        acc = acc + total
    return bsel, kr


def _zero_hist(hist_ref, nbins):
    zeros = jnp.zeros((_L,), jnp.int32)
    for i in range(nbins // _L):
        hist_ref[pl.ds(i * _L, _L)] = zeros


def _sc_thresholds(x, rows, cols, k):
    nvec = cols // _L
    nbins = 256
    unroll = 8
    mesh = plsc.VectorSubcoreMesh(core_axis_name="c", subcore_axis_name="s")
    rows_per_w = rows // _NW

    @functools.partial(
        pl.kernel,
        mesh=mesh,
        out_type=jax.ShapeDtypeStruct((_NW, _L), jnp.int32),
        scratch_types=[
            pltpu.VMEM((cols,), jnp.float32),
            pltpu.VMEM((cols,), jnp.int32),
            pltpu.VMEM((nbins,), jnp.int32),
            pltpu.VMEM((_L,), jnp.int32),
        ],
        compiler_params=pltpu.CompilerParams(needs_layout_passes=False),
    )
    def thresh_kernel(x_hbm, out_hbm, data_v, key_v, hist_v, thr_v):
        c = lax.axis_index("c")
        s = lax.axis_index("s")
        wid = s * _NC + c
        iota = lax.iota(jnp.int32, _L)
        ones = jnp.ones((_L,), jnp.int32)

        thr_v[...] = jnp.zeros((_L,), jnp.int32)

        def row_body(r, _):
            row = wid * rows_per_w + r
            pltpu.sync_copy(x_hbm.at[row], data_v)

            _zero_hist(hist_v, nbins)

            def pass1(i, _):
                for u in range(unroll):
                    j = i * unroll + u
                    v = data_v[pl.ds(j * _L, _L)]
                    kv = _f32_to_key(v)
                    key_v[pl.ds(j * _L, _L)] = kv
                    idx = lax.shift_right_logical(kv, 24)
                    plsc.addupdate_scatter(hist_v, [idx], ones)
                return 0

            lax.fori_loop(0, nvec // unroll, pass1, 0)
            b1, krem = _scan_hist(hist_v, nbins, jnp.int32(k))

            def refine(pref, shift, krem):
                _zero_hist(hist_v, nbins)

                def body(i, _):
                    for u in range(unroll):
                        j = i * unroll + u
                        kv = key_v[pl.ds(j * _L, _L)]
                        m = lax.shift_right_logical(kv, shift + 8) == pref
                        idx = lax.shift_right_logical(kv, shift) & 0xFF
                        plsc.addupdate_scatter(hist_v, [idx], ones, mask=m)
                    return 0

                lax.fori_loop(0, nvec // unroll, body, 0)
                b, krem = _scan_hist(hist_v, nbins, krem)
                return (pref << 8) | b, krem

            pref, krem = refine(b1, 16, krem)
            pref, krem = refine(pref, 8, krem)
            tkey, _ = refine(pref, 0, krem)

            thr_v[...] = jnp.where(iota == r, tkey, thr_v[...])
            return 0

        lax.fori_loop(0, rows_per_w, row_body, 0)
        pltpu.sync_copy(thr_v, out_hbm.at[wid])

    return thresh_kernel(x)


def _mask_block(x_ref, t_ref, o_ref):
    x = x_ref[...]
    t = t_ref[...]
    o_ref[...] = jnp.where(x >= t, x, jnp.float32(0.0))


@jax.jit
def kernel(x):
    rows, cols = x.shape
    k = int(_SR * cols)

    tkeys = _sc_thresholds(x, rows, cols, k)
    rows_per_w = rows // _NW
    tkeys = tkeys[:, :rows_per_w].reshape(rows, 1)
    tbits = jnp.where(tkeys < 0, tkeys ^ _SIGN, ~tkeys)
    thr = lax.bitcast_convert_type(tbits, jnp.float32)

    blk = 16
    grid = (rows // blk,)
    return pl.pallas_call(
        _mask_block,
        grid=grid,
        in_specs=[
            pl.BlockSpec((blk, cols), lambda i: (i, 0)),
            pl.BlockSpec((blk, 1), lambda i: (i, 0)),
        ],
        out_specs=pl.BlockSpec((blk, cols), lambda i: (i, 0)),
        out_shape=jax.ShapeDtypeStruct((rows, cols), x.dtype),
    )(x, thr)


# speedup vs baseline: 3.0461x; 2.9323x over previous
import functools

import jax
import jax.numpy as jnp
import numpy as np
from jax import lax
from jax.experimental import pallas as pl
from jax.experimental.pallas import tpu as pltpu
from jax.experimental.pallas import tpu_sc as plsc

_SR = 0.5

_NC = 2
_NS = 16
_L = 16
_NW = _NC * _NS

_SIGN = np.int32(-2**31)


def _f32_to_key(v):
    y = plsc.bitcast(v, jnp.int32)
    return jnp.where(y < 0, ~y, y ^ _SIGN)


def _scan_hist(hist_ref, nbins, krem):
    nchunks = nbins // _L
    iota = lax.iota(jnp.int32, _L)

    acc = jnp.int32(0)
    kr = krem
    found = jnp.bool_(False)
    bsel = jnp.int32(0)
    for j in range(nchunks - 1, -1, -1):
        bins = hist_ref[pl.ds(j * _L, _L)]
        c = plsc.cumsum(bins)
        total = jnp.max(c)
        excl = c - bins
        rhs = acc + total - krem
        cond = excl <= rhs
        p = jnp.max(plsc.all_reduce_population_count(cond))
        newly = jnp.logical_and(jnp.logical_not(found), p > 0)
        local = p - 1
        c_at = jnp.sum(jnp.where(iota == local, c, 0))
        count_above = acc + total - c_at
        bsel = jnp.where(newly, j * _L + local, bsel)
        kr = jnp.where(newly, krem - count_above, kr)
        found = jnp.logical_or(found, p > 0)
        acc = acc + total
    return bsel, kr


def _zero_hist(hist_ref, nbins):
    zeros = jnp.zeros((_L,), jnp.int32)
    for i in range(nbins // _L):
        hist_ref[pl.ds(i * _L, _L)] = zeros


def _sc_thresholds(x, rows, cols, k):
    nvec = cols // _L
    nbins = 256
    unroll = 8
    mesh = plsc.VectorSubcoreMesh(core_axis_name="c", subcore_axis_name="s")
    rows_per_w = rows // _NW

    @functools.partial(
        pl.kernel,
        mesh=mesh,
        out_type=jax.ShapeDtypeStruct((_NW, _L), jnp.int32),
        scratch_types=[
            pltpu.VMEM((cols,), jnp.float32),
            pltpu.VMEM((cols,), jnp.int32),
            pltpu.VMEM((nbins,), jnp.int32),
            pltpu.VMEM((_L,), jnp.int32),
        ],
        compiler_params=pltpu.CompilerParams(needs_layout_passes=False),
    )
    def thresh_kernel(x_hbm, out_hbm, data_v, key_v, hist_v, thr_v):
        c = lax.axis_index("c")
        s = lax.axis_index("s")
        wid = s * _NC + c
        iota = lax.iota(jnp.int32, _L)
        ones = jnp.ones((_L,), jnp.int32)

        thr_v[...] = jnp.zeros((_L,), jnp.int32)

        def row_body(r, _):
            row = wid * rows_per_w + r
            pltpu.sync_copy(x_hbm.at[row], data_v)

            _zero_hist(hist_v, nbins)

            @plsc.parallel_loop(0, nvec, unroll=unroll)
            def pass1(j):
                v = data_v[pl.ds(j * _L, _L)]
                kv = _f32_to_key(v)
                key_v[pl.ds(j * _L, _L)] = kv
                idx = lax.shift_right_logical(kv, 24)
                plsc.addupdate_scatter(hist_v, [idx], ones)

            b1, krem = _scan_hist(hist_v, nbins, jnp.int32(k))

            def refine(pref, shift, krem):
                _zero_hist(hist_v, nbins)

                @plsc.parallel_loop(0, nvec, unroll=unroll)
                def body(j):
                    kv = key_v[pl.ds(j * _L, _L)]
                    m = lax.shift_right_logical(kv, shift + 8) == pref
                    idx = lax.shift_right_logical(kv, shift) & 0xFF
                    plsc.addupdate_scatter(hist_v, [idx], ones, mask=m)

                b, krem = _scan_hist(hist_v, nbins, krem)
                return (pref << 8) | b, krem

            pref, krem = refine(b1, 16, krem)
            pref, krem = refine(pref, 8, krem)
            tkey, _ = refine(pref, 0, krem)

            thr_v[...] = jnp.where(iota == r, tkey, thr_v[...])
            return 0

        lax.fori_loop(0, rows_per_w, row_body, 0)
        pltpu.sync_copy(thr_v, out_hbm.at[wid])

    return thresh_kernel(x)


def _mask_block(x_ref, t_ref, o_ref):
    x = x_ref[...]
    t = t_ref[...]
    o_ref[...] = jnp.where(x >= t, x, jnp.float32(0.0))


@jax.jit
def kernel(x):
    rows, cols = x.shape
    k = int(_SR * cols)

    tkeys = _sc_thresholds(x, rows, cols, k)
    rows_per_w = rows // _NW
    tkeys = tkeys[:, :rows_per_w].reshape(rows, 1)
    tbits = jnp.where(tkeys < 0, tkeys ^ _SIGN, ~tkeys)
    thr = lax.bitcast_convert_type(tbits, jnp.float32)

    blk = 16
    grid = (rows // blk,)
    return pl.pallas_call(
        _mask_block,
        grid=grid,
        in_specs=[
            pl.BlockSpec((blk, cols), lambda i: (i, 0)),
            pl.BlockSpec((blk, 1), lambda i: (i, 0)),
        ],
        out_specs=pl.BlockSpec((blk, cols), lambda i: (i, 0)),
        out_shape=jax.ShapeDtypeStruct((rows, cols), x.dtype),
    )(x, thr)
